# baseline (speedup 1.0000x reference)
import jax
import jax.numpy as jnp
from jax import lax
from jax.experimental import pallas as pl
from jax.experimental.pallas import tpu as pltpu
from jax.experimental.pallas import tpu_sc as plsc

N = 25000
D = 128
E = 400000
NC = 2
NS = 16
NW = NC * NS
LANES = 16
CHUNK = 128
NCHUNKS = E // CHUNK
LCH = 112
LPAD = 2 * LCH
SPLIT = 12504
GARBAGE = 12504
STRIPE = 784
ACC_ROWS = 12512
CAP = 12544
PBUF = CAP + 256
LISTS = (4 * NW + 1) * CAP


def _partition_one(ei3, lists_out, in_s, in_d, in_sb, in_db, lo_s, lo_d,
                   hi_s, hi_d, cbuf, psem, wid):
    nchunks_tile = (NCHUNKS - wid + NW - 1) // NW
    pltpu.async_copy(ei3.at[0, wid], in_sb, psem)
    pltpu.async_copy(ei3.at[1, wid], in_db, psem)

    def chunk_body(i, carry):
        n_lo, n_hi = carry
        j = wid + i * NW
        pltpu.make_async_copy(ei3.at[0, j], in_sb, psem).wait()
        pltpu.make_async_copy(ei3.at[1, j], in_db, psem).wait()
        for k in range(CHUNK // LANES):
            in_s[pl.ds(k * LANES, LANES)] = in_sb[pl.ds(k * LANES, LANES)]
            in_d[pl.ds(k * LANES, LANES)] = in_db[pl.ds(k * LANES, LANES)]

        @pl.when(i + 1 < nchunks_tile)
        def _():
            pltpu.async_copy(ei3.at[0, j + NW], in_sb, psem)
            pltpu.async_copy(ei3.at[1, j + NW], in_db, psem)

        def group_body(g, carry2):
            nl, nh = carry2
            dvec = in_d[pl.ds(g * LANES, LANES)]
            svec = in_s[pl.ds(g * LANES, LANES)]
            io = lax.iota(jnp.int32, LANES)
            comp_ls = svec
            comp_ld = dvec
            comp_hs = svec
            comp_hd = dvec
            c_lo = jnp.int32(0)
            c_hi = jnp.int32(0)
            for i_ in range(LANES):
                d_i = dvec[i_]
                s_i = svec[i_]
                is_lo = d_i < SPLIT
                sel_lo = io == c_lo
                comp_ls = jnp.where(sel_lo, s_i, comp_ls)
                comp_ld = jnp.where(sel_lo, d_i, comp_ld)
                sel_hi = io == c_hi
                comp_hs = jnp.where(sel_hi, s_i, comp_hs)
                comp_hd = jnp.where(sel_hi, d_i - SPLIT, comp_hd)
                ilo = jnp.where(is_lo, 1, 0).astype(jnp.int32)
                c_lo = c_lo + ilo
                c_hi = c_hi + 1 - ilo
            lo_s[pl.ds(nl, LANES)] = comp_ls
            lo_d[pl.ds(nl, LANES)] = comp_ld
            hi_s[pl.ds(nh, LANES)] = comp_hs
            hi_d[pl.ds(nh, LANES)] = comp_hd
            return nl + c_lo, nh + c_hi

        return lax.fori_loop(0, CHUNK // LANES, group_body, (n_lo, n_hi))

    n_lo, n_hi = lax.fori_loop(0, nchunks_tile, chunk_body,
                               (jnp.int32(0), jnp.int32(0)))

    zvec = jnp.zeros((LANES,), jnp.int32)
    gvec = jnp.full((LANES,), GARBAGE, jnp.int32)
    for g in range(LPAD // LANES):
        lo_s[pl.ds(n_lo + g * LANES, LANES)] = zvec
        lo_d[pl.ds(n_lo + g * LANES, LANES)] = gvec
        hi_s[pl.ds(n_hi + g * LANES, LANES)] = zvec
        hi_d[pl.ds(n_hi + g * LANES, LANES)] = gvec

    pltpu.sync_copy(lo_s.at[pl.ds(0, CAP)],
                    lists_out.at[pl.ds(wid * CAP, CAP)])
    pltpu.sync_copy(lo_d.at[pl.ds(0, CAP)],
                    lists_out.at[pl.ds((2 * NW + wid) * CAP, CAP)])
    pltpu.sync_copy(hi_s.at[pl.ds(0, CAP)],
                    lists_out.at[pl.ds((NW + wid) * CAP, CAP)])
    pltpu.sync_copy(hi_d.at[pl.ds(0, CAP)],
                    lists_out.at[pl.ds((3 * NW + wid) * CAP, CAP)])
    cbuf[...] = jnp.full((LANES,), ((n_lo + LPAD - 1) // LPAD) * 2,
                         jnp.int32)
    pltpu.sync_copy(cbuf,
                    lists_out.at[pl.ds(4 * NW * CAP + wid * LANES, LANES)])
    cbuf[...] = jnp.full((LANES,), ((n_hi + LPAD - 1) // LPAD) * 2,
                         jnp.int32)
    pltpu.sync_copy(cbuf, lists_out.at[
        pl.ds(4 * NW * CAP + (NW + wid) * LANES, LANES)])


def _partition3_body(e1, e2, e3, o1, o2, o3, in_s, in_d, in_sb, in_db,
                     lo_s, lo_d, hi_s, hi_d, cbuf, psem):
    c = lax.axis_index("c")
    s = lax.axis_index("s")
    wid = c * NS + s
    for ei3, out in ((e1, o1), (e2, o2), (e3, o3)):
        _partition_one(ei3, out, in_s, in_d, in_sb, in_db, lo_s, lo_d,
                       hi_s, hi_d, cbuf, psem, wid)


_SC_MESH = plsc.VectorSubcoreMesh(core_axis_name="c", subcore_axis_name="s")

_partition3 = pl.kernel(
    _partition3_body,
    out_type=(jax.ShapeDtypeStruct((LISTS,), jnp.int32),) * 3,
    mesh=_SC_MESH,
    scratch_types=[
        pltpu.VMEM((CHUNK,), jnp.int32),
        pltpu.VMEM((CHUNK,), jnp.int32),
        pltpu.VMEM((CHUNK,), jnp.int32),
        pltpu.VMEM((CHUNK,), jnp.int32),
        pltpu.VMEM((PBUF,), jnp.int32),
        pltpu.VMEM((PBUF,), jnp.int32),
        pltpu.VMEM((PBUF,), jnp.int32),
        pltpu.VMEM((PBUF,), jnp.int32),
        pltpu.VMEM((LANES,), jnp.int32),
        pltpu.SemaphoreType.DMA,
    ],
)


def _zero_accum(accum, zsrc, s):
    def zero_body(i, _):
        pltpu.sync_copy(zsrc, accum.at[pl.ds(s * STRIPE + i * 8, 8), :])
        return ()

    lax.fori_loop(0, jnp.where(s == NS - 1, (ACC_ROWS - 15 * STRIPE) // 8,
                               STRIPE // 8), zero_body, ())


def _writeback(accum, out, base, c, s):
    r0 = s * STRIPE

    @pl.when(s < NS - 1)
    def _():
        pltpu.sync_copy(accum.at[pl.ds(r0, STRIPE), :],
                        out.at[pl.ds(base + r0, STRIPE), :])

    for cc, nlast in ((0, SPLIT - 15 * STRIPE), (1, N - SPLIT - 15 * STRIPE)):
        @pl.when((s == NS - 1) & (c == cc))
        def _(nlast=nlast):
            pltpu.sync_copy(accum.at[pl.ds(r0, nlast), :],
                            out.at[pl.ds(base + r0, nlast), :])


def _row_meta(lists, r, di0):
    pltpu.sync_copy(lists.at[pl.ds(4 * NW * CAP + r * LANES, LANES)],
                    di0.at[pl.ds(0, LANES)])
    return di0[pl.ds(0, LANES)][0]


def _idx_copy(dst, src):
    for k in range(LCH // LANES):
        dst[pl.ds(k * LANES, LANES)] = src[pl.ds(k * LANES, LANES)]


def _agg_pass(table, lists, agg_out, si0, si1, di0, di1, sib, dib,
              rows0, rows1, gsem0, gsem1, isem, ssem0, ssem1, accum, c, s):
    base = c * SPLIT
    _zero_accum(accum, rows0.at[pl.ds(0, 8), :], s)
    plsc.subcore_barrier()

    def process_row(r):
        n = _row_meta(lists, r, di0)
        rs = r * CAP
        rd = (2 * NW + r) * CAP

        @pl.when(n >= 2)
        def _():
            pltpu.sync_copy(lists.at[pl.ds(rs, LCH)], si0)
            pltpu.sync_copy(lists.at[pl.ds(rd, LCH)], di0)
            pltpu.sync_copy(lists.at[pl.ds(rs + LCH, LCH)], si1)
            pltpu.sync_copy(lists.at[pl.ds(rd + LCH, LCH)], di1)
            pltpu.async_copy(table.at[si0], rows0, gsem0)
            pltpu.async_copy(table.at[si1], rows1, gsem1)

            @pl.when(n >= 4)
            def _():
                pltpu.async_copy(lists.at[pl.ds(rs + 2 * LCH, LCH)], sib,
                                 isem)
                pltpu.async_copy(lists.at[pl.ds(rd + 2 * LCH, LCH)], dib,
                                 isem)

            def pair_body(i2, _):
                pltpu.make_async_copy(table.at[si0], rows0, gsem0).wait()
                sca = pltpu.async_copy(rows0, accum.at[di0], ssem0,
                                       add=True)
                pltpu.make_async_copy(table.at[si1], rows1, gsem1).wait()
                scb = pltpu.async_copy(rows1, accum.at[di1], ssem1,
                                       add=True)
                pltpu.make_async_copy(
                    lists.at[pl.ds(rs + (2 * i2 + 2) * LCH, LCH)], sib,
                    isem).wait()
                pltpu.make_async_copy(
                    lists.at[pl.ds(rd + (2 * i2 + 2) * LCH, LCH)], dib,
                    isem).wait()
                sca.wait()
                _idx_copy(si0, sib)
                _idx_copy(di0, dib)
                pltpu.async_copy(table.at[si0], rows0, gsem0)
                pltpu.async_copy(lists.at[pl.ds(rs + (2 * i2 + 3) * LCH,
                                                LCH)], sib, isem)
                pltpu.async_copy(lists.at[pl.ds(rd + (2 * i2 + 3) * LCH,
                                                LCH)], dib, isem)

                pltpu.make_async_copy(
                    lists.at[pl.ds(rs + (2 * i2 + 3) * LCH, LCH)], sib,
                    isem).wait()
                pltpu.make_async_copy(
                    lists.at[pl.ds(rd + (2 * i2 + 3) * LCH, LCH)], dib,
                    isem).wait()
                scb.wait()
                _idx_copy(si1, sib)
                _idx_copy(di1, dib)
                pltpu.async_copy(table.at[si1], rows1, gsem1)

                @pl.when(2 * i2 + 4 < n)
                def _():
                    pltpu.async_copy(lists.at[pl.ds(rs + (2 * i2 + 4) * LCH,
                                                    LCH)], sib, isem)
                    pltpu.async_copy(lists.at[pl.ds(rd + (2 * i2 + 4) * LCH,
                                                    LCH)], dib, isem)
                return ()

            lax.fori_loop(0, n // 2 - 1, pair_body, ())
            pltpu.make_async_copy(table.at[si0], rows0, gsem0).wait()
            pltpu.sync_copy(rows0, accum.at[di0], add=True)
            pltpu.make_async_copy(table.at[si1], rows1, gsem1).wait()
            pltpu.sync_copy(rows1, accum.at[di1], add=True)

    process_row(c * (2 * NS) + 2 * s)
    process_row(c * (2 * NS) + 2 * s + 1)
    plsc.subcore_barrier()
    _writeback(accum, agg_out, base, c, s)


def _zero_rows0(rows0):
    for col in range(0, D, LANES):
        for r in range(8):
            rows0[r, pl.ds(col, LANES)] = jnp.zeros((LANES,), jnp.float32)


def _agg3_body(tcell, tdrug, lr, ld, lc, out_r, out_d, out_c,
               si0, si1, di0, di1, sib, dib, rows0, rows1,
               gsem0, gsem1, isem, ssem0, ssem1, accum):
    c = lax.axis_index("c")
    s = lax.axis_index("s")
    first = True
    for table, lists, out in ((tcell, lr, out_r), (tdrug, ld, out_d),
                              (tcell, lc, out_c)):
        if not first:
            plsc.subcore_barrier()
        first = False
        _zero_rows0(rows0)
        _agg_pass(table, lists, out, si0, si1, di0, di1, sib, dib,
                  rows0, rows1, gsem0, gsem1, isem, ssem0, ssem1, accum,
                  c, s)


_agg3 = pl.kernel(
    _agg3_body,
    out_type=(jax.ShapeDtypeStruct((N, D), jnp.float32),) * 3,
    mesh=_SC_MESH,
    scratch_types=[
        pltpu.VMEM((LCH,), jnp.int32),
        pltpu.VMEM((LCH,), jnp.int32),
        pltpu.VMEM((LCH,), jnp.int32),
        pltpu.VMEM((LCH,), jnp.int32),
        pltpu.VMEM((LCH,), jnp.int32),
        pltpu.VMEM((LCH,), jnp.int32),
        pltpu.VMEM((LCH, D), jnp.float32),
        pltpu.VMEM((LCH, D), jnp.float32),
        pltpu.SemaphoreType.DMA,
        pltpu.SemaphoreType.DMA,
        pltpu.SemaphoreType.DMA,
        pltpu.SemaphoreType.DMA,
        pltpu.SemaphoreType.DMA,
        pltpu.VMEM_SHARED((ACC_ROWS, D), jnp.float32),
    ],
)


def _cnt_pass(lists, cnt_out, di0, di1, dib, ones, isem, accum, c, s):
    base = c * SPLIT
    _zero_accum(accum, ones.at[pl.ds(LCH, 8), :], s)
    plsc.subcore_barrier()

    def process_row(r):
        n = _row_meta(lists, r, di0)
        rd = (2 * NW + r) * CAP

        @pl.when(n >= 2)
        def _():
            pltpu.sync_copy(lists.at[pl.ds(rd, LCH)], di0)
            pltpu.sync_copy(lists.at[pl.ds(rd + LCH, LCH)], di1)

            @pl.when(n >= 4)
            def _():
                pltpu.async_copy(lists.at[pl.ds(rd + 2 * LCH, LCH)], dib,
                                 isem)

            def pair_body(i2, _):
                pltpu.make_async_copy(
                    lists.at[pl.ds(rd + (2 * i2 + 2) * LCH, LCH)], dib,
                    isem).wait()
                pltpu.sync_copy(ones.at[pl.ds(0, LCH), :], accum.at[di0],
                                add=True)
                _idx_copy(di0, dib)
                pltpu.async_copy(lists.at[pl.ds(rd + (2 * i2 + 3) * LCH,
                                                LCH)], dib, isem)
                pltpu.make_async_copy(
                    lists.at[pl.ds(rd + (2 * i2 + 3) * LCH, LCH)], dib,
                    isem).wait()
                pltpu.sync_copy(ones.at[pl.ds(0, LCH), :], accum.at[di1],
                                add=True)
                _idx_copy(di1, dib)

                @pl.when(2 * i2 + 4 < n)
                def _():
                    pltpu.async_copy(lists.at[pl.ds(rd + (2 * i2 + 4) * LCH,
                                                    LCH)], dib, isem)
                return ()

            lax.fori_loop(0, n // 2 - 1, pair_body, ())
            pltpu.sync_copy(ones.at[pl.ds(0, LCH), :], accum.at[di0],
                            add=True)
            pltpu.sync_copy(ones.at[pl.ds(0, LCH), :], accum.at[di1],
                            add=True)

    process_row(c * (2 * NS) + 2 * s)
    process_row(c * (2 * NS) + 2 * s + 1)
    plsc.subcore_barrier()
    _writeback(accum, cnt_out, base, c, s)


def _cnt3_body(lr, ld, lc, out_r, out_d, out_c, di0, di1, dib, ones,
               isem, accum):
    c = lax.axis_index("c")
    s = lax.axis_index("s")
    for col in range(0, D, LANES):
        for r in range(LCH):
            ones[r, pl.ds(col, LANES)] = jnp.ones((LANES,), jnp.float32)
        for r in range(LCH, LCH + 8):
            ones[r, pl.ds(col, LANES)] = jnp.zeros((LANES,), jnp.float32)
    first = True
    for lists, out in ((lr, out_r), (ld, out_d), (lc, out_c)):
        if not first:
            plsc.subcore_barrier()
        first = False
        _cnt_pass(lists, out, di0, di1, dib, ones, isem, accum, c, s)


_cnt3 = pl.kernel(
    _cnt3_body,
    out_type=(jax.ShapeDtypeStruct((N, D), jnp.float32),) * 3,
    mesh=_SC_MESH,
    scratch_types=[
        pltpu.VMEM((LCH,), jnp.int32),
        pltpu.VMEM((LCH,), jnp.int32),
        pltpu.VMEM((LCH,), jnp.int32),
        pltpu.VMEM((LCH + 8, D), jnp.float32),
        pltpu.SemaphoreType.DMA,
        pltpu.VMEM_SHARED((ACC_ROWS, D), jnp.float32),
    ],
)

ROWS_BLK = 1000


def _dense2_body(agg1, cnt1, agg2, cnt2, x, W1, W2, Wr, b, out):
    m1 = agg1[...] / jnp.maximum(cnt1[:, 0:1], 1.0)
    m2 = agg2[...] / jnp.maximum(cnt2[:, 0:1], 1.0)
    acc = jnp.dot(m1, W1[...], preferred_element_type=jnp.float32)
    acc += jnp.dot(m2, W2[...], preferred_element_type=jnp.float32)
    acc += jnp.dot(x[...], Wr[...], preferred_element_type=jnp.float32)
    out[...] = jnp.maximum(acc + b[...], 0.0)


def _dense1_body(agg1, cnt1, x, W1, Wr, b, out):
    m1 = agg1[...] / jnp.maximum(cnt1[:, 0:1], 1.0)
    acc = jnp.dot(m1, W1[...], preferred_element_type=jnp.float32)
    acc += jnp.dot(x[...], Wr[...], preferred_element_type=jnp.float32)
    out[...] = jnp.maximum(acc + b[...], 0.0)


def _rows_spec():
    return pl.BlockSpec((ROWS_BLK, D), lambda i: (i, 0))


def _w_spec():
    return pl.BlockSpec((D, D), lambda i: (0, 0))


def _b_spec():
    return pl.BlockSpec((1, D), lambda i: (0, 0))


_dense2 = pl.pallas_call(
    _dense2_body,
    grid=(N // ROWS_BLK,),
    in_specs=[_rows_spec(), _rows_spec(), _rows_spec(), _rows_spec(),
              _rows_spec(), _w_spec(), _w_spec(), _w_spec(), _b_spec()],
    out_specs=_rows_spec(),
    out_shape=jax.ShapeDtypeStruct((N, D), jnp.float32),
)

_dense1 = pl.pallas_call(
    _dense1_body,
    grid=(N // ROWS_BLK,),
    in_specs=[_rows_spec(), _rows_spec(), _rows_spec(), _w_spec(),
              _w_spec(), _b_spec()],
    out_specs=_rows_spec(),
    out_shape=jax.ShapeDtypeStruct((N, D), jnp.float32),
)


def kernel(x_cell, x_drug, ei_resp, ei_csim, ei_dsim,
           Wl1_resp, Wr1_resp, b1_resp, Wl1_csim, Wr1_csim, b1_csim,
           Wl1_dsim, Wr1_dsim, b1_dsim,
           Wl2_resp, Wr2_resp, b2_resp, Wl2_csim, Wr2_csim, b2_csim,
           Wl2_dsim, Wr2_dsim, b2_dsim):
    ei_resp = ei_resp.astype(jnp.int32).reshape(2, NCHUNKS, CHUNK)
    ei_csim = ei_csim.astype(jnp.int32).reshape(2, NCHUNKS, CHUNK)
    ei_dsim = ei_dsim.astype(jnp.int32).reshape(2, NCHUNKS, CHUNK)

    pr, pd, pc = _partition3(ei_resp, ei_dsim, ei_csim)
    cnt_r, cnt_d, cnt_c = _cnt3(pr, pd, pc)

    agg_r, agg_d, agg_c = _agg3(x_cell, x_drug, pr, pd, pc)
    h_drug = _dense2(agg_r, cnt_r, agg_d, cnt_d, x_drug,
                     0.5 * Wl1_resp, 0.5 * Wl1_dsim,
                     0.5 * (Wr1_resp + Wr1_dsim),
                     (0.5 * (b1_resp + b1_dsim)).reshape(1, D))
    h_cell = _dense1(agg_c, cnt_c, x_cell, Wl1_csim, Wr1_csim,
                     b1_csim.reshape(1, D))

    agg_r2, agg_d2, agg_c2 = _agg3(h_cell, h_drug, pr, pd, pc)
    o_drug = _dense2(agg_r2, cnt_r, agg_d2, cnt_d, h_drug,
                     0.5 * Wl2_resp, 0.5 * Wl2_dsim,
                     0.5 * (Wr2_resp + Wr2_dsim),
                     (0.5 * (b2_resp + b2_dsim)).reshape(1, D))
    o_cell = _dense1(agg_c2, cnt_c, h_cell, Wl2_csim, Wr2_csim,
                     b2_csim.reshape(1, D))
    return (o_cell, o_drug)


# speedup vs baseline: 1.0268x; 1.0268x over previous
import jax
import jax.numpy as jnp
from jax import lax
from jax.experimental import pallas as pl
from jax.experimental.pallas import tpu as pltpu
from jax.experimental.pallas import tpu_sc as plsc

N = 25000
D = 128
E = 400000
NC = 2
NS = 16
NW = NC * NS
LANES = 16
CHUNK = 128
NCHUNKS = E // CHUNK
LCH = 112
LPAD = 2 * LCH
SPLIT = 12504
GARBAGE = 12504
STRIPE = 784
ACC_ROWS = 12512
CAP = 12544
PBUF = CAP + 256
LISTS = (4 * NW + 1) * CAP


def _partition_one(ei3, lists_out, in_s, in_d, in_sb, in_db, lo_s, lo_d,
                   hi_s, hi_d, cbuf, psem, wid):
    nchunks_tile = (NCHUNKS - wid + NW - 1) // NW
    pltpu.async_copy(ei3.at[0, wid], in_sb, psem)
    pltpu.async_copy(ei3.at[1, wid], in_db, psem)

    def chunk_body(i, carry):
        n_lo, n_hi = carry
        j = wid + i * NW
        pltpu.make_async_copy(ei3.at[0, j], in_sb, psem).wait()
        pltpu.make_async_copy(ei3.at[1, j], in_db, psem).wait()
        for k in range(CHUNK // LANES):
            in_s[pl.ds(k * LANES, LANES)] = in_sb[pl.ds(k * LANES, LANES)]
            in_d[pl.ds(k * LANES, LANES)] = in_db[pl.ds(k * LANES, LANES)]

        @pl.when(i + 1 < nchunks_tile)
        def _():
            pltpu.async_copy(ei3.at[0, j + NW], in_sb, psem)
            pltpu.async_copy(ei3.at[1, j + NW], in_db, psem)

        def group_body(g, carry2):
            nl, nh = carry2
            dvec = in_d[pl.ds(g * LANES, LANES)]
            svec = in_s[pl.ds(g * LANES, LANES)]
            io = lax.iota(jnp.int32, LANES)
            comp_ls = svec
            comp_ld = dvec
            comp_hs = svec
            comp_hd = dvec
            c_lo = jnp.int32(0)
            c_hi = jnp.int32(0)
            for i_ in range(LANES):
                d_i = dvec[i_]
                s_i = svec[i_]
                is_lo = d_i < SPLIT
                sel_lo = io == c_lo
                comp_ls = jnp.where(sel_lo, s_i, comp_ls)
                comp_ld = jnp.where(sel_lo, d_i, comp_ld)
                sel_hi = io == c_hi
                comp_hs = jnp.where(sel_hi, s_i, comp_hs)
                comp_hd = jnp.where(sel_hi, d_i - SPLIT, comp_hd)
                ilo = jnp.where(is_lo, 1, 0).astype(jnp.int32)
                c_lo = c_lo + ilo
                c_hi = c_hi + 1 - ilo
            lo_s[pl.ds(nl, LANES)] = comp_ls
            lo_d[pl.ds(nl, LANES)] = comp_ld
            hi_s[pl.ds(nh, LANES)] = comp_hs
            hi_d[pl.ds(nh, LANES)] = comp_hd
            return nl + c_lo, nh + c_hi

        return lax.fori_loop(0, CHUNK // LANES, group_body, (n_lo, n_hi))

    n_lo, n_hi = lax.fori_loop(0, nchunks_tile, chunk_body,
                               (jnp.int32(0), jnp.int32(0)))

    zvec = jnp.zeros((LANES,), jnp.int32)
    gvec = jnp.full((LANES,), GARBAGE, jnp.int32)
    for g in range(LPAD // LANES):
        lo_s[pl.ds(n_lo + g * LANES, LANES)] = zvec
        lo_d[pl.ds(n_lo + g * LANES, LANES)] = gvec
        hi_s[pl.ds(n_hi + g * LANES, LANES)] = zvec
        hi_d[pl.ds(n_hi + g * LANES, LANES)] = gvec

    pltpu.sync_copy(lo_s.at[pl.ds(0, CAP)],
                    lists_out.at[pl.ds(wid * CAP, CAP)])
    pltpu.sync_copy(lo_d.at[pl.ds(0, CAP)],
                    lists_out.at[pl.ds((2 * NW + wid) * CAP, CAP)])
    pltpu.sync_copy(hi_s.at[pl.ds(0, CAP)],
                    lists_out.at[pl.ds((NW + wid) * CAP, CAP)])
    pltpu.sync_copy(hi_d.at[pl.ds(0, CAP)],
                    lists_out.at[pl.ds((3 * NW + wid) * CAP, CAP)])
    cbuf[...] = jnp.full((LANES,), ((n_lo + LPAD - 1) // LPAD) * 2,
                         jnp.int32)
    pltpu.sync_copy(cbuf,
                    lists_out.at[pl.ds(4 * NW * CAP + wid * LANES, LANES)])
    cbuf[...] = jnp.full((LANES,), ((n_hi + LPAD - 1) // LPAD) * 2,
                         jnp.int32)
    pltpu.sync_copy(cbuf, lists_out.at[
        pl.ds(4 * NW * CAP + (NW + wid) * LANES, LANES)])


def _partition3_body(e1, e2, e3, o1, o2, o3, in_s, in_d, in_sb, in_db,
                     lo_s, lo_d, hi_s, hi_d, cbuf, psem):
    c = lax.axis_index("c")
    s = lax.axis_index("s")
    wid = c * NS + s
    for ei3, out in ((e1, o1), (e2, o2), (e3, o3)):
        _partition_one(ei3, out, in_s, in_d, in_sb, in_db, lo_s, lo_d,
                       hi_s, hi_d, cbuf, psem, wid)


_SC_MESH = plsc.VectorSubcoreMesh(core_axis_name="c", subcore_axis_name="s")

_partition3 = pl.kernel(
    _partition3_body,
    out_type=(jax.ShapeDtypeStruct((LISTS,), jnp.int32),) * 3,
    mesh=_SC_MESH,
    scratch_types=[
        pltpu.VMEM((CHUNK,), jnp.int32),
        pltpu.VMEM((CHUNK,), jnp.int32),
        pltpu.VMEM((CHUNK,), jnp.int32),
        pltpu.VMEM((CHUNK,), jnp.int32),
        pltpu.VMEM((PBUF,), jnp.int32),
        pltpu.VMEM((PBUF,), jnp.int32),
        pltpu.VMEM((PBUF,), jnp.int32),
        pltpu.VMEM((PBUF,), jnp.int32),
        pltpu.VMEM((LANES,), jnp.int32),
        pltpu.SemaphoreType.DMA,
    ],
)


def _zero_accum(accum, zsrc, s):
    def zero_body(i, _):
        pltpu.sync_copy(zsrc, accum.at[pl.ds(s * STRIPE + i * 8, 8), :])
        return ()

    lax.fori_loop(0, jnp.where(s == NS - 1, (ACC_ROWS - 15 * STRIPE) // 8,
                               STRIPE // 8), zero_body, ())


def _writeback(accum, out, base, c, s):
    r0 = s * STRIPE

    @pl.when(s < NS - 1)
    def _():
        pltpu.sync_copy(accum.at[pl.ds(r0, STRIPE), :],
                        out.at[pl.ds(base + r0, STRIPE), :])

    for cc, nlast in ((0, SPLIT - 15 * STRIPE), (1, N - SPLIT - 15 * STRIPE)):
        @pl.when((s == NS - 1) & (c == cc))
        def _(nlast=nlast):
            pltpu.sync_copy(accum.at[pl.ds(r0, nlast), :],
                            out.at[pl.ds(base + r0, nlast), :])


def _row_meta(lists, r, di0):
    pltpu.sync_copy(lists.at[pl.ds(4 * NW * CAP + r * LANES, LANES)],
                    di0.at[pl.ds(0, LANES)])
    return di0[pl.ds(0, LANES)][0]


def _idx_copy(dst, src):
    for k in range(LCH // LANES):
        dst[pl.ds(k * LANES, LANES)] = src[pl.ds(k * LANES, LANES)]


def _agg_pass(table, lists, agg_out, si0, si1, di0, di1, sib, dib,
              rows0, rows1, gsem0, gsem1, isem, ssem0, ssem1, accum, c, s):
    base = c * SPLIT
    _zero_accum(accum, rows0.at[pl.ds(0, 8), :], s)
    plsc.subcore_barrier()

    def process_row(r):
        n = _row_meta(lists, r, di0)
        rs = r * CAP
        rd = (2 * NW + r) * CAP

        @pl.when(n >= 2)
        def _():
            pltpu.sync_copy(lists.at[pl.ds(rs, LCH)], si0)
            pltpu.sync_copy(lists.at[pl.ds(rd, LCH)], di0)
            pltpu.sync_copy(lists.at[pl.ds(rs + LCH, LCH)], si1)
            pltpu.sync_copy(lists.at[pl.ds(rd + LCH, LCH)], di1)
            pltpu.async_copy(table.at[si0], rows0, gsem0)
            pltpu.async_copy(table.at[si1], rows1, gsem1)

            @pl.when(n >= 4)
            def _():
                pltpu.async_copy(lists.at[pl.ds(rs + 2 * LCH, LCH)], sib,
                                 isem)
                pltpu.async_copy(lists.at[pl.ds(rd + 2 * LCH, LCH)], dib,
                                 isem)

            def pair_body(i2, _):
                pltpu.make_async_copy(table.at[si0], rows0, gsem0).wait()
                sca = pltpu.async_copy(rows0, accum.at[di0], ssem0,
                                       add=True)
                pltpu.make_async_copy(table.at[si1], rows1, gsem1).wait()
                scb = pltpu.async_copy(rows1, accum.at[di1], ssem1,
                                       add=True)
                pltpu.make_async_copy(
                    lists.at[pl.ds(rs + (2 * i2 + 2) * LCH, LCH)], sib,
                    isem).wait()
                pltpu.make_async_copy(
                    lists.at[pl.ds(rd + (2 * i2 + 2) * LCH, LCH)], dib,
                    isem).wait()
                sca.wait()
                _idx_copy(si0, sib)
                _idx_copy(di0, dib)
                pltpu.async_copy(table.at[si0], rows0, gsem0)
                pltpu.async_copy(lists.at[pl.ds(rs + (2 * i2 + 3) * LCH,
                                                LCH)], sib, isem)
                pltpu.async_copy(lists.at[pl.ds(rd + (2 * i2 + 3) * LCH,
                                                LCH)], dib, isem)

                pltpu.make_async_copy(
                    lists.at[pl.ds(rs + (2 * i2 + 3) * LCH, LCH)], sib,
                    isem).wait()
                pltpu.make_async_copy(
                    lists.at[pl.ds(rd + (2 * i2 + 3) * LCH, LCH)], dib,
                    isem).wait()
                scb.wait()
                _idx_copy(si1, sib)
                _idx_copy(di1, dib)
                pltpu.async_copy(table.at[si1], rows1, gsem1)

                @pl.when(2 * i2 + 4 < n)
                def _():
                    pltpu.async_copy(lists.at[pl.ds(rs + (2 * i2 + 4) * LCH,
                                                    LCH)], sib, isem)
                    pltpu.async_copy(lists.at[pl.ds(rd + (2 * i2 + 4) * LCH,
                                                    LCH)], dib, isem)
                return ()

            lax.fori_loop(0, n // 2 - 1, pair_body, ())
            pltpu.make_async_copy(table.at[si0], rows0, gsem0).wait()
            pltpu.sync_copy(rows0, accum.at[di0], add=True)
            pltpu.make_async_copy(table.at[si1], rows1, gsem1).wait()
            pltpu.sync_copy(rows1, accum.at[di1], add=True)

    process_row(c * (2 * NS) + 2 * s)
    process_row(c * (2 * NS) + 2 * s + 1)
    plsc.subcore_barrier()
    _writeback(accum, agg_out, base, c, s)


def _zero_rows0(rows0):
    for col in range(0, D, LANES):
        for r in range(8):
            rows0[r, pl.ds(col, LANES)] = jnp.zeros((LANES,), jnp.float32)


def _agg3_body(tcell, tdrug, lr, ld, lc, out_r, out_d, out_c,
               si0, si1, di0, di1, sib, dib, rows0, rows1,
               gsem0, gsem1, isem, ssem0, ssem1, accum):
    c = lax.axis_index("c")
    s = lax.axis_index("s")
    first = True
    for table, lists, out in ((tcell, lr, out_r), (tdrug, ld, out_d),
                              (tcell, lc, out_c)):
        if not first:
            plsc.subcore_barrier()
        first = False
        _zero_rows0(rows0)
        _agg_pass(table, lists, out, si0, si1, di0, di1, sib, dib,
                  rows0, rows1, gsem0, gsem1, isem, ssem0, ssem1, accum,
                  c, s)


_agg3 = pl.kernel(
    _agg3_body,
    out_type=(jax.ShapeDtypeStruct((N, D), jnp.float32),) * 3,
    mesh=_SC_MESH,
    scratch_types=[
        pltpu.VMEM((LCH,), jnp.int32),
        pltpu.VMEM((LCH,), jnp.int32),
        pltpu.VMEM((LCH,), jnp.int32),
        pltpu.VMEM((LCH,), jnp.int32),
        pltpu.VMEM((LCH,), jnp.int32),
        pltpu.VMEM((LCH,), jnp.int32),
        pltpu.VMEM((LCH, D), jnp.float32),
        pltpu.VMEM((LCH, D), jnp.float32),
        pltpu.SemaphoreType.DMA,
        pltpu.SemaphoreType.DMA,
        pltpu.SemaphoreType.DMA,
        pltpu.SemaphoreType.DMA,
        pltpu.SemaphoreType.DMA,
        pltpu.VMEM_SHARED((ACC_ROWS, D), jnp.float32),
    ],
)


def _cnt_pass(lists, cnt_out, di0, di1, ones, isem, accum, c, s):
    base = c * SPLIT
    _zero_accum(accum, ones.at[pl.ds(LCH, 8), :], s)
    plsc.subcore_barrier()

    def process_row(r):
        n = _row_meta(lists, r, di0)
        rd = (2 * NW + r) * CAP

        @pl.when(n >= 2)
        def _():
            pltpu.sync_copy(lists.at[pl.ds(rd, LCH)], di0)
            pltpu.sync_copy(lists.at[pl.ds(rd + LCH, LCH)], di1)

            def pair_body(i2, _):
                pltpu.sync_copy(ones.at[pl.ds(0, LCH), :], accum.at[di0],
                                add=True)
                ia = pltpu.async_copy(
                    lists.at[pl.ds(rd + (2 * i2 + 2) * LCH, LCH)], di0, isem)
                pltpu.sync_copy(ones.at[pl.ds(0, LCH), :], accum.at[di1],
                                add=True)
                ib = pltpu.async_copy(
                    lists.at[pl.ds(rd + (2 * i2 + 3) * LCH, LCH)], di1, isem)
                ia.wait()
                ib.wait()
                return ()

            lax.fori_loop(0, n // 2 - 1, pair_body, ())
            pltpu.sync_copy(ones.at[pl.ds(0, LCH), :], accum.at[di0],
                            add=True)
            pltpu.sync_copy(ones.at[pl.ds(0, LCH), :], accum.at[di1],
                            add=True)

    process_row(c * (2 * NS) + 2 * s)
    process_row(c * (2 * NS) + 2 * s + 1)
    plsc.subcore_barrier()
    _writeback(accum, cnt_out, base, c, s)


def _cnt3_body(lr, ld, lc, out_r, out_d, out_c, di0, di1, ones,
               isem, accum):
    c = lax.axis_index("c")
    s = lax.axis_index("s")
    for col in range(0, D, LANES):
        for r in range(LCH):
            ones[r, pl.ds(col, LANES)] = jnp.ones((LANES,), jnp.float32)
        for r in range(LCH, LCH + 8):
            ones[r, pl.ds(col, LANES)] = jnp.zeros((LANES,), jnp.float32)
    first = True
    for lists, out in ((lr, out_r), (ld, out_d), (lc, out_c)):
        if not first:
            plsc.subcore_barrier()
        first = False
        _cnt_pass(lists, out, di0, di1, ones, isem, accum, c, s)


_cnt3 = pl.kernel(
    _cnt3_body,
    out_type=(jax.ShapeDtypeStruct((N, D), jnp.float32),) * 3,
    mesh=_SC_MESH,
    scratch_types=[
        pltpu.VMEM((LCH,), jnp.int32),
        pltpu.VMEM((LCH,), jnp.int32),
        pltpu.VMEM((LCH + 8, D), jnp.float32),
        pltpu.SemaphoreType.DMA,
        pltpu.VMEM_SHARED((ACC_ROWS, D), jnp.float32),
    ],
)

ROWS_BLK = 1000


def _dense2_body(agg1, cnt1, agg2, cnt2, x, W1, W2, Wr, b, out):
    m1 = agg1[...] / jnp.maximum(cnt1[:, 0:1], 1.0)
    m2 = agg2[...] / jnp.maximum(cnt2[:, 0:1], 1.0)
    acc = jnp.dot(m1, W1[...], preferred_element_type=jnp.float32)
    acc += jnp.dot(m2, W2[...], preferred_element_type=jnp.float32)
    acc += jnp.dot(x[...], Wr[...], preferred_element_type=jnp.float32)
    out[...] = jnp.maximum(acc + b[...], 0.0)


def _dense1_body(agg1, cnt1, x, W1, Wr, b, out):
    m1 = agg1[...] / jnp.maximum(cnt1[:, 0:1], 1.0)
    acc = jnp.dot(m1, W1[...], preferred_element_type=jnp.float32)
    acc += jnp.dot(x[...], Wr[...], preferred_element_type=jnp.float32)
    out[...] = jnp.maximum(acc + b[...], 0.0)


def _rows_spec():
    return pl.BlockSpec((ROWS_BLK, D), lambda i: (i, 0))


def _w_spec():
    return pl.BlockSpec((D, D), lambda i: (0, 0))


def _b_spec():
    return pl.BlockSpec((1, D), lambda i: (0, 0))


_dense2 = pl.pallas_call(
    _dense2_body,
    grid=(N // ROWS_BLK,),
    in_specs=[_rows_spec(), _rows_spec(), _rows_spec(), _rows_spec(),
              _rows_spec(), _w_spec(), _w_spec(), _w_spec(), _b_spec()],
    out_specs=_rows_spec(),
    out_shape=jax.ShapeDtypeStruct((N, D), jnp.float32),
)

_dense1 = pl.pallas_call(
    _dense1_body,
    grid=(N // ROWS_BLK,),
    in_specs=[_rows_spec(), _rows_spec(), _rows_spec(), _w_spec(),
              _w_spec(), _b_spec()],
    out_specs=_rows_spec(),
    out_shape=jax.ShapeDtypeStruct((N, D), jnp.float32),
)


def kernel(x_cell, x_drug, ei_resp, ei_csim, ei_dsim,
           Wl1_resp, Wr1_resp, b1_resp, Wl1_csim, Wr1_csim, b1_csim,
           Wl1_dsim, Wr1_dsim, b1_dsim,
           Wl2_resp, Wr2_resp, b2_resp, Wl2_csim, Wr2_csim, b2_csim,
           Wl2_dsim, Wr2_dsim, b2_dsim):
    ei_resp = ei_resp.astype(jnp.int32).reshape(2, NCHUNKS, CHUNK)
    ei_csim = ei_csim.astype(jnp.int32).reshape(2, NCHUNKS, CHUNK)
    ei_dsim = ei_dsim.astype(jnp.int32).reshape(2, NCHUNKS, CHUNK)

    pr, pd, pc = _partition3(ei_resp, ei_dsim, ei_csim)
    cnt_r, cnt_d, cnt_c = _cnt3(pr, pd, pc)

    agg_r, agg_d, agg_c = _agg3(x_cell, x_drug, pr, pd, pc)
    h_drug = _dense2(agg_r, cnt_r, agg_d, cnt_d, x_drug,
                     0.5 * Wl1_resp, 0.5 * Wl1_dsim,
                     0.5 * (Wr1_resp + Wr1_dsim),
                     (0.5 * (b1_resp + b1_dsim)).reshape(1, D))
    h_cell = _dense1(agg_c, cnt_c, x_cell, Wl1_csim, Wr1_csim,
                     b1_csim.reshape(1, D))

    agg_r2, agg_d2, agg_c2 = _agg3(h_cell, h_drug, pr, pd, pc)
    o_drug = _dense2(agg_r2, cnt_r, agg_d2, cnt_d, h_drug,
                     0.5 * Wl2_resp, 0.5 * Wl2_dsim,
                     0.5 * (Wr2_resp + Wr2_dsim),
                     (0.5 * (b2_resp + b2_dsim)).reshape(1, D))
    o_cell = _dense1(agg_c2, cnt_c, h_cell, Wl2_csim, Wr2_csim,
                     b2_csim.reshape(1, D))
    return (o_cell, o_drug)
